# baseline (speedup 1.0000x reference)
import functools

import numpy as np
import jax
import jax.numpy as jnp
from jax.experimental import pallas as pl
from jax.experimental.pallas import tpu as pltpu

_OUT_LEN = 256


@functools.lru_cache(maxsize=None)
def _block_weights_np(length: int, out_len: int, nblk: int) -> np.ndarray:
    w = np.zeros((length, out_len), dtype=np.float32)
    for i in range(out_len):
        start = (i * length) // out_len
        end = -(-((i + 1) * length) // out_len)
        w[start:end, i] = 1.0 / float(end - start)
    kb, nb = length // nblk, out_len // nblk
    blocks = np.stack(
        [w[j * kb : (j + 1) * kb, j * nb : (j + 1) * nb] for j in range(nblk)]
    )
    return blocks.astype(np.dtype("bfloat16"))


def _pool_mm_kernel(x_ref, w_ref, o_ref):
    kb = x_ref.shape[1] // 2
    nb = o_ref.shape[1] // 2
    xb = x_ref[...].astype(jnp.bfloat16)
    o_ref[:, :nb] = jnp.dot(
        xb[:, :kb], w_ref[0], preferred_element_type=jnp.float32
    ).astype(o_ref.dtype)
    o_ref[:, nb:] = jnp.dot(
        xb[:, kb:], w_ref[1], preferred_element_type=jnp.float32
    ).astype(o_ref.dtype)


def kernel(x):
    b, c, length = x.shape
    out_len = _OUT_LEN
    m = b * c
    x2 = x.reshape(m, length)

    w = jnp.asarray(_block_weights_np(length, out_len, 2))

    tm = 4096
    grid = (pl.cdiv(m, tm),)

    out2 = pl.pallas_call(
        _pool_mm_kernel,
        out_shape=jax.ShapeDtypeStruct((m, out_len), x.dtype),
        grid=grid,
        in_specs=[
            pl.BlockSpec((tm, length), lambda i: (i, 0)),
            pl.BlockSpec(w.shape, lambda i: (0, 0, 0)),
        ],
        out_specs=pl.BlockSpec((tm, out_len), lambda i: (i, 0)),
        compiler_params=pltpu.CompilerParams(
            dimension_semantics=("parallel",),
        ),
    )(x2, w)

    return out2.reshape(b, c, out_len)


# speedup vs baseline: 1.0297x; 1.0297x over previous
import functools

import numpy as np
import jax
import jax.numpy as jnp
from jax.experimental import pallas as pl
from jax.experimental.pallas import tpu as pltpu

_OUT_LEN = 256


@functools.lru_cache(maxsize=None)
def _block_weights_np(length: int, out_len: int, nblk: int) -> np.ndarray:
    w = np.zeros((length, out_len), dtype=np.float32)
    for i in range(out_len):
        start = (i * length) // out_len
        end = -(-((i + 1) * length) // out_len)
        w[start:end, i] = 1.0 / float(end - start)
    kb, nb = length // nblk, out_len // nblk
    blocks = np.stack(
        [w[j * kb : (j + 1) * kb, j * nb : (j + 1) * nb] for j in range(nblk)]
    )
    return blocks


def _pool_mm_kernel(x_ref, w_ref, o_ref):
    kb = x_ref.shape[1] // 2
    nb = o_ref.shape[1] // 2
    x = x_ref[...]
    o_ref[:, :nb] = jnp.dot(
        x[:, :kb], w_ref[0], preferred_element_type=jnp.float32
    ).astype(o_ref.dtype)
    o_ref[:, nb:] = jnp.dot(
        x[:, kb:], w_ref[1], preferred_element_type=jnp.float32
    ).astype(o_ref.dtype)


def kernel(x):
    b, c, length = x.shape
    out_len = _OUT_LEN
    m = b * c
    x2 = x.reshape(m, length)

    w = jnp.asarray(_block_weights_np(length, out_len, 2))

    tm = 2048
    grid = (pl.cdiv(m, tm),)

    out2 = pl.pallas_call(
        _pool_mm_kernel,
        out_shape=jax.ShapeDtypeStruct((m, out_len), x.dtype),
        grid=grid,
        in_specs=[
            pl.BlockSpec((tm, length), lambda i: (i, 0)),
            pl.BlockSpec(w.shape, lambda i: (0, 0, 0)),
        ],
        out_specs=pl.BlockSpec((tm, out_len), lambda i: (i, 0)),
        compiler_params=pltpu.CompilerParams(
            dimension_semantics=("parallel",),
        ),
    )(x2, w)

    return out2.reshape(b, c, out_len)
